# baseline (device time: 24443 ns/iter reference)
import jax
import jax.numpy as jnp
from jax import lax
from jax.experimental import pallas as pl
from jax.experimental.pallas import tpu as pltpu

CHUNKS = (32, 64, 96, 128, 96, 48, 32, 16)
C = len(CHUNKS)
OFFS = tuple(sum(CHUNKS[:i]) for i in range(C))


def kernel(A, B):
    M, K = A.shape
    K2, N = B.shape
    assert K == K2
    Mh = M // 2
    assert sum(CHUNKS) == Mh

    A = A.astype(jnp.bfloat16)
    B = B.astype(jnp.bfloat16)

    def body(a_hbm, b_hbm, out_hbm,
             a_v, b_v, xsend, xrecv, ystage, yrecv,
             in_sems, out_sems,
             xsend_sems, xrecv_sems, ysend_sems, yrecv_sems):
        my_x = lax.axis_index("x")
        my_y = lax.axis_index("y")
        peer_x = (1 - my_x, my_y)
        peer_y = (my_x, 1 - my_y)
        my_base = my_y * Mh
        other_base = (1 - my_y) * Mh

        barrier_sem = pltpu.get_barrier_semaphore()
        for peer in (peer_x, peer_y):
            pl.semaphore_signal(
                barrier_sem, inc=1,
                device_id=peer, device_id_type=pl.DeviceIdType.MESH,
            )
        cp_b = pltpu.make_async_copy(b_hbm, b_v, in_sems.at[0])
        cp_b.start()
        cp_a = pltpu.make_async_copy(
            a_hbm.at[pl.ds(my_base, Mh), :], a_v, in_sems.at[1])
        cp_a.start()
        cp_b.wait()
        cp_a.wait()

        x_rdmas = []
        for c in range(C):
            off, rc = OFFS[c], CHUNKS[c]
            part_c = jnp.dot(a_v[pl.ds(off, rc), :], b_v[...],
                             preferred_element_type=jnp.float32)
            xsend[pl.ds(off, rc), :] = part_c.astype(jnp.bfloat16)
            if c == 0:
                pl.semaphore_wait(barrier_sem, 2)
            rdma = pltpu.make_async_remote_copy(
                src_ref=xsend.at[pl.ds(off, rc), :],
                dst_ref=xrecv.at[pl.ds(off, rc), :],
                send_sem=xsend_sems.at[c],
                recv_sem=xrecv_sems.at[c],
                device_id=peer_x,
                device_id_type=pl.DeviceIdType.MESH,
            )
            rdma.start()
            x_rdmas.append(rdma)

        y_rdmas = []
        out_cps = []
        for c in range(C):
            off, rc = OFFS[c], CHUNKS[c]
            x_rdmas[c].wait_recv()
            ystage[pl.ds(off, rc), :] = (
                xsend[pl.ds(off, rc), :] + xrecv[pl.ds(off, rc), :])
            rdma = pltpu.make_async_remote_copy(
                src_ref=ystage.at[pl.ds(off, rc), :],
                dst_ref=yrecv.at[pl.ds(off, rc), :],
                send_sem=ysend_sems.at[c],
                recv_sem=yrecv_sems.at[c],
                device_id=peer_y,
                device_id_type=pl.DeviceIdType.MESH,
            )
            rdma.start()
            y_rdmas.append(rdma)
            cp = pltpu.make_async_copy(
                ystage.at[pl.ds(off, rc), :],
                out_hbm.at[pl.ds(my_base + off, rc), :],
                out_sems.at[c])
            cp.start()
            out_cps.append(cp)

        for c in range(C):
            off, rc = OFFS[c], CHUNKS[c]
            y_rdmas[c].wait_recv()
            cp = pltpu.make_async_copy(
                yrecv.at[pl.ds(off, rc), :],
                out_hbm.at[pl.ds(other_base + off, rc), :],
                out_sems.at[C + c])
            cp.start()
            out_cps.append(cp)

        for cp in out_cps:
            cp.wait()
        for c in range(C):
            x_rdmas[c].wait_send()
            y_rdmas[c].wait_send()

    return pl.pallas_call(
        body,
        out_shape=jax.ShapeDtypeStruct((M, N), jnp.bfloat16),
        in_specs=[
            pl.BlockSpec(memory_space=pl.ANY),
            pl.BlockSpec(memory_space=pl.ANY),
        ],
        out_specs=pl.BlockSpec(memory_space=pl.ANY),
        scratch_shapes=[
            pltpu.VMEM((Mh, K), jnp.bfloat16),
            pltpu.VMEM((K, N), jnp.bfloat16),
            pltpu.VMEM((Mh, N), jnp.bfloat16),
            pltpu.VMEM((Mh, N), jnp.bfloat16),
            pltpu.VMEM((Mh, N), jnp.bfloat16),
            pltpu.VMEM((Mh, N), jnp.bfloat16),
            pltpu.SemaphoreType.DMA((2,)),
            pltpu.SemaphoreType.DMA((2 * C,)),
            pltpu.SemaphoreType.DMA((C,)),
            pltpu.SemaphoreType.DMA((C,)),
            pltpu.SemaphoreType.DMA((C,)),
            pltpu.SemaphoreType.DMA((C,)),
        ],
        compiler_params=pltpu.CompilerParams(collective_id=0),
    )(A, B)


# device time: 24305 ns/iter; 1.0057x vs baseline; 1.0057x over previous
import jax
import jax.numpy as jnp
from jax import lax
from jax.experimental import pallas as pl
from jax.experimental.pallas import tpu as pltpu

CHUNKS = (32, 64, 96, 128, 96, 48, 32, 16)
C = len(CHUNKS)
OFFS = tuple(sum(CHUNKS[:i]) for i in range(C))


def kernel(A, B):
    M, K = A.shape
    K2, N = B.shape
    assert K == K2
    Mh = M // 2
    assert sum(CHUNKS) == Mh

    A = A.astype(jnp.bfloat16)

    def body(a_hbm, b_hbm, out_hbm,
             a_v, b_v, xsend, xrecv, ystage, yrecv,
             in_sems, out_sems,
             xsend_sems, xrecv_sems, ysend_sems, yrecv_sems):
        my_x = lax.axis_index("x")
        my_y = lax.axis_index("y")
        peer_x = (1 - my_x, my_y)
        peer_y = (my_x, 1 - my_y)
        my_base = my_y * Mh
        other_base = (1 - my_y) * Mh

        barrier_sem = pltpu.get_barrier_semaphore()
        for peer in (peer_x, peer_y):
            pl.semaphore_signal(
                barrier_sem, inc=1,
                device_id=peer, device_id_type=pl.DeviceIdType.MESH,
            )
        cp_b = pltpu.make_async_copy(b_hbm, b_v, in_sems.at[0])
        cp_b.start()
        cp_a = pltpu.make_async_copy(
            a_hbm.at[pl.ds(my_base, Mh), :], a_v, in_sems.at[1])
        cp_a.start()
        cp_b.wait()
        cp_a.wait()

        b_bf16 = b_v[...].astype(jnp.bfloat16)

        x_rdmas = []
        for c in range(C):
            off, rc = OFFS[c], CHUNKS[c]
            part_c = jnp.dot(a_v[pl.ds(off, rc), :], b_bf16,
                             preferred_element_type=jnp.float32)
            xsend[pl.ds(off, rc), :] = part_c.astype(jnp.bfloat16)
            if c == 0:
                pl.semaphore_wait(barrier_sem, 2)
            rdma = pltpu.make_async_remote_copy(
                src_ref=xsend.at[pl.ds(off, rc), :],
                dst_ref=xrecv.at[pl.ds(off, rc), :],
                send_sem=xsend_sems.at[c],
                recv_sem=xrecv_sems.at[c],
                device_id=peer_x,
                device_id_type=pl.DeviceIdType.MESH,
            )
            rdma.start()
            x_rdmas.append(rdma)

        y_rdmas = []
        out_cps = []
        for c in range(C):
            off, rc = OFFS[c], CHUNKS[c]
            x_rdmas[c].wait_recv()
            ystage[pl.ds(off, rc), :] = (
                xsend[pl.ds(off, rc), :] + xrecv[pl.ds(off, rc), :])
            rdma = pltpu.make_async_remote_copy(
                src_ref=ystage.at[pl.ds(off, rc), :],
                dst_ref=yrecv.at[pl.ds(off, rc), :],
                send_sem=ysend_sems.at[c],
                recv_sem=yrecv_sems.at[c],
                device_id=peer_y,
                device_id_type=pl.DeviceIdType.MESH,
            )
            rdma.start()
            y_rdmas.append(rdma)
            cp = pltpu.make_async_copy(
                ystage.at[pl.ds(off, rc), :],
                out_hbm.at[pl.ds(my_base + off, rc), :],
                out_sems.at[c])
            cp.start()
            out_cps.append(cp)

        for c in range(C):
            off, rc = OFFS[c], CHUNKS[c]
            y_rdmas[c].wait_recv()
            cp = pltpu.make_async_copy(
                yrecv.at[pl.ds(off, rc), :],
                out_hbm.at[pl.ds(other_base + off, rc), :],
                out_sems.at[C + c])
            cp.start()
            out_cps.append(cp)

        for cp in out_cps:
            cp.wait()
        for c in range(C):
            x_rdmas[c].wait_send()
            y_rdmas[c].wait_send()

    return pl.pallas_call(
        body,
        out_shape=jax.ShapeDtypeStruct((M, N), jnp.bfloat16),
        in_specs=[
            pl.BlockSpec(memory_space=pl.ANY),
            pl.BlockSpec(memory_space=pl.ANY),
        ],
        out_specs=pl.BlockSpec(memory_space=pl.ANY),
        scratch_shapes=[
            pltpu.VMEM((Mh, K), jnp.bfloat16),
            pltpu.VMEM((K, N), jnp.float32),
            pltpu.VMEM((Mh, N), jnp.bfloat16),
            pltpu.VMEM((Mh, N), jnp.bfloat16),
            pltpu.VMEM((Mh, N), jnp.bfloat16),
            pltpu.VMEM((Mh, N), jnp.bfloat16),
            pltpu.SemaphoreType.DMA((2,)),
            pltpu.SemaphoreType.DMA((2 * C,)),
            pltpu.SemaphoreType.DMA((C,)),
            pltpu.SemaphoreType.DMA((C,)),
            pltpu.SemaphoreType.DMA((C,)),
            pltpu.SemaphoreType.DMA((C,)),
        ],
        compiler_params=pltpu.CompilerParams(collective_id=0),
    )(A, B)


# device time: 23053 ns/iter; 1.0603x vs baseline; 1.0543x over previous
import jax
import jax.numpy as jnp
from jax import lax
from jax.experimental import pallas as pl
from jax.experimental.pallas import tpu as pltpu

CHUNKS = (64, 64, 64, 64, 64, 64, 64, 64)
C = len(CHUNKS)
OFFS = tuple(sum(CHUNKS[:i]) for i in range(C))


def kernel(A, B):
    M, K = A.shape
    K2, N = B.shape
    assert K == K2
    Mh = M // 2
    assert sum(CHUNKS) == Mh

    A = A.astype(jnp.bfloat16)

    def body(a_hbm, b_hbm, out_hbm,
             a_v, b_v, xsend, xrecv, ystage, yrecv,
             in_sems, out_sems,
             xsend_sems, xrecv_sems, ysend_sems, yrecv_sems):
        my_x = lax.axis_index("x")
        my_y = lax.axis_index("y")
        peer_x = (1 - my_x, my_y)
        peer_y = (my_x, 1 - my_y)
        my_base = my_y * Mh
        other_base = (1 - my_y) * Mh

        barrier_sem = pltpu.get_barrier_semaphore()
        for peer in (peer_x, peer_y):
            pl.semaphore_signal(
                barrier_sem, inc=1,
                device_id=peer, device_id_type=pl.DeviceIdType.MESH,
            )
        cp_b = pltpu.make_async_copy(b_hbm, b_v, in_sems.at[0])
        cp_b.start()
        cp_a = pltpu.make_async_copy(
            a_hbm.at[pl.ds(my_base, Mh), :], a_v, in_sems.at[1])
        cp_a.start()
        cp_b.wait()
        cp_a.wait()

        b_bf16 = b_v[...].astype(jnp.bfloat16)

        x_rdmas = []
        for c in range(C):
            off, rc = OFFS[c], CHUNKS[c]
            part_c = jnp.dot(a_v[pl.ds(off, rc), :], b_bf16,
                             preferred_element_type=jnp.float32)
            xsend[pl.ds(off, rc), :] = part_c.astype(jnp.bfloat16)
            if c == 0:
                pl.semaphore_wait(barrier_sem, 2)
            rdma = pltpu.make_async_remote_copy(
                src_ref=xsend.at[pl.ds(off, rc), :],
                dst_ref=xrecv.at[pl.ds(off, rc), :],
                send_sem=xsend_sems.at[c],
                recv_sem=xrecv_sems.at[c],
                device_id=peer_x,
                device_id_type=pl.DeviceIdType.MESH,
            )
            rdma.start()
            x_rdmas.append(rdma)

        y_rdmas = []
        out_cps = []
        for c in range(C):
            off, rc = OFFS[c], CHUNKS[c]
            x_rdmas[c].wait_recv()
            ystage[pl.ds(off, rc), :] = (
                xsend[pl.ds(off, rc), :] + xrecv[pl.ds(off, rc), :])
            rdma = pltpu.make_async_remote_copy(
                src_ref=ystage.at[pl.ds(off, rc), :],
                dst_ref=yrecv.at[pl.ds(off, rc), :],
                send_sem=ysend_sems.at[c],
                recv_sem=yrecv_sems.at[c],
                device_id=peer_y,
                device_id_type=pl.DeviceIdType.MESH,
            )
            rdma.start()
            y_rdmas.append(rdma)
            cp = pltpu.make_async_copy(
                ystage.at[pl.ds(off, rc), :],
                out_hbm.at[pl.ds(my_base + off, rc), :],
                out_sems.at[c])
            cp.start()
            out_cps.append(cp)

        for c in range(C):
            off, rc = OFFS[c], CHUNKS[c]
            y_rdmas[c].wait_recv()
            cp = pltpu.make_async_copy(
                yrecv.at[pl.ds(off, rc), :],
                out_hbm.at[pl.ds(other_base + off, rc), :],
                out_sems.at[C + c])
            cp.start()
            out_cps.append(cp)

        for cp in out_cps:
            cp.wait()
        for c in range(C):
            x_rdmas[c].wait_send()
            y_rdmas[c].wait_send()

    return pl.pallas_call(
        body,
        out_shape=jax.ShapeDtypeStruct((M, N), jnp.bfloat16),
        in_specs=[
            pl.BlockSpec(memory_space=pl.ANY),
            pl.BlockSpec(memory_space=pl.ANY),
        ],
        out_specs=pl.BlockSpec(memory_space=pl.ANY),
        scratch_shapes=[
            pltpu.VMEM((Mh, K), jnp.bfloat16),
            pltpu.VMEM((K, N), jnp.float32),
            pltpu.VMEM((Mh, N), jnp.bfloat16),
            pltpu.VMEM((Mh, N), jnp.bfloat16),
            pltpu.VMEM((Mh, N), jnp.bfloat16),
            pltpu.VMEM((Mh, N), jnp.bfloat16),
            pltpu.SemaphoreType.DMA((2,)),
            pltpu.SemaphoreType.DMA((2 * C,)),
            pltpu.SemaphoreType.DMA((C,)),
            pltpu.SemaphoreType.DMA((C,)),
            pltpu.SemaphoreType.DMA((C,)),
            pltpu.SemaphoreType.DMA((C,)),
        ],
        compiler_params=pltpu.CompilerParams(collective_id=0),
    )(A, B)
